# baseline (speedup 1.0000x reference)
import functools

import jax
import jax.numpy as jnp
from jax import lax
from jax.experimental import pallas as pl
from jax.experimental.pallas import tpu as pltpu
from jax.experimental.pallas import tpu_sc as plsc

NC = 2
NS = 16
NW = NC * NS
CH = 80
L = 16
NB = 12
GA = 5


def _sc_mesh():
    return plsc.VectorSubcoreMesh(core_axis_name="c", subcore_axis_name="s")


_SC_PARAMS = pltpu.CompilerParams(use_tc_tiling_on_sc=False)


def _deg_kernel(n_acc, cm):
    zrows = n_acc // NS
    per = cm * CH
    G = cm // NB

    @functools.partial(
        pl.kernel,
        out_type=jax.ShapeDtypeStruct((NC, n_acc), jnp.float32),
        mesh=_sc_mesh(),
        compiler_params=_SC_PARAMS,
        scratch_types=[
            pltpu.VMEM((per,), jnp.int32),
            pltpu.VMEM((CH,), jnp.float32),
            pltpu.VMEM((zrows,), jnp.float32),
            pltpu.VMEM_SHARED((n_acc,), jnp.float32),
        ] + [pltpu.SemaphoreType.DMA] * NB,
    )
    def deg_kernel(edge_hbm, out_hbm, didx, ones_v, zbuf, acc, *ssem):
        c = lax.axis_index("c")
        s = lax.axis_index("s")
        wid = c * NS + s

        def _fill(i, _):
            zbuf[pl.ds(i * L, L)] = jnp.zeros((L,), jnp.float32)
            ones_v[pl.ds(lax.rem(i, CH // L) * L, L)] = (
                jnp.ones((L,), jnp.float32))
            return 0

        lax.fori_loop(0, zrows // L, _fill, 0)
        pltpu.sync_copy(zbuf, acc.at[pl.ds(s * zrows, zrows)])
        pltpu.sync_copy(edge_hbm.at[1, pl.ds(wid * per, per)], didx)
        plsc.subcore_barrier()

        def _start(i, b):
            pltpu.async_copy(ones_v, acc.at[didx.at[pl.ds(i * CH, CH)]],
                             ssem[b], add=True)

        def _wait(b):
            pltpu.make_async_copy(ones_v, acc.at[didx.at[pl.ds(0, CH)]],
                                  ssem[b]).wait()

        def _group(g, _):
            for b in range(NB):
                @pl.when(g > 0)
                def _():
                    _wait(b)
                _start(g * NB + b, b)
            return 0

        lax.fori_loop(0, G, _group, 0)
        for i in range(G * NB, cm):
            if i >= NB:
                _wait(i % NB)
            _start(i, i % NB)
        for j in range(max(0, cm - NB), cm):
            _wait(j % NB)
        plsc.subcore_barrier()
        pltpu.sync_copy(acc.at[pl.ds(s * zrows, zrows)],
                        out_hbm.at[c, pl.ds(s * zrows, zrows)])

    return deg_kernel


def _agg_kernel(n_acc, cm, d):
    zrows = n_acc // NS
    per = cm * CH
    G, rem = cm // NB, cm % NB
    assert rem >= GA and cm >= NB

    @functools.partial(
        pl.kernel,
        out_type=jax.ShapeDtypeStruct((NC, n_acc, d), jnp.float32),
        mesh=_sc_mesh(),
        compiler_params=_SC_PARAMS,
        scratch_types=[
            pltpu.VMEM((per,), jnp.int32),
            pltpu.VMEM((per,), jnp.int32),
        ] + [pltpu.VMEM((CH, d), jnp.float32)] * NB + [
            pltpu.VMEM((zrows, d), jnp.float32),
            pltpu.VMEM_SHARED((n_acc, d), jnp.float32),
        ] + [pltpu.SemaphoreType.DMA] * (2 * NB),
    )
    def agg_kernel(edge_hbm, table_hbm, out_hbm, sidx, didx, *rest):
        rows = rest[:NB]
        zbuf = rest[NB]
        acc = rest[NB + 1]
        gsem = rest[NB + 2:2 * NB + 2]
        ssem = rest[2 * NB + 2:]
        c = lax.axis_index("c")
        s = lax.axis_index("s")
        wid = c * NS + s

        def _fill(i, _):
            zbuf[i, :] = jnp.zeros((L,), jnp.float32)
            return 0

        lax.fori_loop(0, zrows, _fill, 0)
        pltpu.sync_copy(zbuf, acc.at[pl.ds(s * zrows, zrows)])
        pltpu.sync_copy(edge_hbm.at[0, pl.ds(wid * per, per)], sidx)
        pltpu.sync_copy(edge_hbm.at[1, pl.ds(wid * per, per)], didx)
        plsc.subcore_barrier()

        def _gather(i, b):
            pltpu.async_copy(table_hbm.at[sidx.at[pl.ds(i * CH, CH)]],
                             rows[b], gsem[b])

        def _gwait(b):
            pltpu.make_async_copy(table_hbm.at[sidx.at[pl.ds(0, CH)]],
                                  rows[b], gsem[b]).wait()

        def _scat(i, b):
            pltpu.async_copy(rows[b], acc.at[didx.at[pl.ds(i * CH, CH)]],
                             ssem[b], add=True)

        def _swait(b):
            pltpu.make_async_copy(rows[b], acc.at[didx.at[pl.ds(0, CH)]],
                                  ssem[b]).wait()

        for j in range(GA):
            _gather(j, j)

        def _group(g, _):
            for b in range(NB):
                i = g * NB + b
                tb = (b + GA) % NB
                if b + GA >= NB:
                    _swait(tb)
                    _gather(i + GA, tb)
                else:
                    @pl.when(g > 0)
                    def _():
                        _swait(tb)
                    _gather(i + GA, tb)
                _gwait(b)
                _scat(i, b)
            return 0

        lax.fori_loop(0, G, _group, 0)
        for i in range(G * NB, cm):
            b = i % NB
            tb = (b + GA) % NB
            if i + GA < cm:
                _swait(tb)
                _gather(i + GA, tb)
            _gwait(b)
            _scat(i, b)
        for j in range(max(0, cm - NB), cm):
            _swait(j % NB)
        plsc.subcore_barrier()
        pltpu.sync_copy(acc.at[pl.ds(s * zrows, zrows)],
                        out_hbm.at[c, pl.ds(s * zrows, zrows)])

    return agg_kernel


def _tc_mm(x_flat, w1_bd):
    rows, k = x_flat.shape
    d_out = w1_bd.shape[1]

    def body(x_ref, w_ref, h_ref):
        h_ref[...] = jnp.dot(x_ref[...], w_ref[...],
                             preferred_element_type=jnp.float32,
                       precision=lax.Precision.HIGHEST)

    return pl.pallas_call(
        body,
        out_shape=jax.ShapeDtypeStruct((rows, d_out), jnp.float32),
    )(x_flat, w1_bd)


def _tc_dinv(deg_view):
    rows = deg_view.shape[1]

    def body(deg_ref, dinv_ref):
        dinv_ref[...] = lax.rsqrt(1.0 + deg_ref[0] + deg_ref[1])

    return pl.pallas_call(
        body,
        out_shape=jax.ShapeDtypeStruct((rows, 128), jnp.float32),
    )(deg_view)


def _tc_scale(h1_ff, dinv_ff):
    shape = h1_ff.shape

    def body(h_ref, d_ref, hpre_ref, self1_ref):
        hp = h_ref[...] * d_ref[...]
        hpre_ref[...] = hp
        self1_ref[...] = hp * d_ref[...]

    return pl.pallas_call(
        body,
        out_shape=(jax.ShapeDtypeStruct(shape, jnp.float32),
                   jax.ShapeDtypeStruct(shape, jnp.float32)),
    )(h1_ff, dinv_ff)


def _tc_mid(agg1v, dinv_ff, self1_ff, b1t):
    shape = self1_ff.shape

    def body(agg_ref, d_ref, self1_ref, b1_ref, h2pre_ref, self2_ref):
        d = d_ref[...]
        h = d * (agg_ref[0] + agg_ref[1]) + self1_ref[...] + b1_ref[...]
        h = jnp.maximum(h, 0.0)
        h2p = h * d
        h2pre_ref[...] = h2p
        self2_ref[...] = h2p * d

    return pl.pallas_call(
        body,
        out_shape=(jax.ShapeDtypeStruct(shape, jnp.float32),
                   jax.ShapeDtypeStruct(shape, jnp.float32)),
    )(agg1v, dinv_ff, self1_ff, b1t)


def _tc_fin(agg2v, dinv_ff, self2_ff, w2_bd, b2t, blk, n_out, d_fin):
    rows = self2_ff.shape[0]

    def body(agg_ref, d_ref, self2_ref, w_ref, b_ref, out_ref):
        aggf = d_ref[...] * (agg_ref[0] + agg_ref[1]) + self2_ref[...]
        o = jnp.dot(aggf, w_ref[...], preferred_element_type=jnp.float32,
                       precision=lax.Precision.HIGHEST)
        o = o + b_ref[...]
        m = jnp.max(o, axis=1, keepdims=True)
        e = jnp.exp(o - m)
        s8 = jnp.kron(jnp.eye(blk, dtype=jnp.float32),
                      jnp.ones((blk, 1), jnp.float32))
        b8 = jnp.kron(jnp.eye(blk, dtype=jnp.float32),
                      jnp.ones((1, blk), jnp.float32))
        s = jnp.dot(e, s8, preferred_element_type=jnp.float32,
                       precision=lax.Precision.HIGHEST)
        ls = jnp.log(s)
        out_ff = (o - m) - jnp.dot(ls, b8,
                                   preferred_element_type=jnp.float32,
                                   precision=lax.Precision.HIGHEST)
        out_ref[...] = out_ff

    return pl.pallas_call(
        body,
        out_shape=jax.ShapeDtypeStruct((rows, blk * blk), jnp.float32),
    )(agg2v, dinv_ff, self2_ff, w2_bd, b2t)


def kernel(x, edge_index, W1, b1, W2, b2):
    n, d_in = x.shape
    e = edge_index.shape[1]
    d_hid = W1.shape[1]
    d_out = W2.shape[1]
    blk = 128 // d_hid

    n_acc = ((n + 1 + NS * L - 1) // (NS * L)) * (NS * L)
    frows = n_acc // blk
    e_pad = ((e + NW * CH - 1) // (NW * CH)) * (NW * CH)
    cm = e_pad // (NW * CH)

    edges = edge_index
    if e_pad != e:
        pad = jnp.concatenate(
            [jnp.zeros((1, e_pad - e), jnp.int32),
             jnp.full((1, e_pad - e), n, jnp.int32)])
        edges = jnp.concatenate([edge_index, pad], axis=1)

    x_flat = jnp.pad(x.reshape(n // blk, blk * d_in),
                     ((0, frows - n // blk), (0, 0)))
    eye = jnp.eye(blk, dtype=jnp.float32)
    w1_bd = jnp.kron(eye, W1)
    w2_pad = jnp.concatenate(
        [W2, jnp.full((d_hid, blk - d_out), 0.0)], axis=1)
    w2_bd = jnp.kron(eye, w2_pad)
    b1t = jnp.tile(b1, blk).reshape(1, 128)
    b2t = jnp.tile(jnp.concatenate(
        [b2, jnp.full((blk - d_out,), -1e9, jnp.float32)]), blk).reshape(1, -1)

    h1_ff = _tc_mm(x_flat, w1_bd)
    deg_part = _deg_kernel(n_acc, cm)(edges)
    dinv_n = _tc_dinv(deg_part.reshape(2, n_acc // 128, 128))
    dinv_ff = jnp.repeat(dinv_n.reshape(n_acc), d_hid).reshape(frows, 128)

    hpre_ff, self1_ff = _tc_scale(h1_ff, dinv_ff)
    agg1 = _agg_kernel(n_acc, cm, d_hid)(edges, hpre_ff.reshape(n_acc, d_hid))
    h2pre_ff, self2_ff = _tc_mid(agg1.reshape(2, frows, 128), dinv_ff,
                                 self1_ff, b1t)
    agg2 = _agg_kernel(n_acc, cm, d_hid)(edges, h2pre_ff.reshape(n_acc, d_hid))
    out_ff = _tc_fin(agg2.reshape(2, frows, 128), dinv_ff, self2_ff,
                     w2_bd, b2t, blk, n, d_out)
    return out_ff.reshape(n_acc, blk)[:n, :d_out]


# speedup vs baseline: 1.0326x; 1.0326x over previous
import functools

import jax
import jax.numpy as jnp
from jax import lax
from jax.experimental import pallas as pl
from jax.experimental.pallas import tpu as pltpu
from jax.experimental.pallas import tpu_sc as plsc

NC = 2
NS = 16
NW = NC * NS
CH = 80
L = 16
NB = 12
GA = 5


def _sc_mesh():
    return plsc.VectorSubcoreMesh(core_axis_name="c", subcore_axis_name="s")


_SC_PARAMS = pltpu.CompilerParams(use_tc_tiling_on_sc=False)


def _deg_kernel(n_acc, cm):
    zrows = n_acc // NS
    per = cm * CH
    G = cm // NB

    @functools.partial(
        pl.kernel,
        out_type=jax.ShapeDtypeStruct((NC, n_acc), jnp.float32),
        mesh=_sc_mesh(),
        compiler_params=_SC_PARAMS,
        scratch_types=[
            pltpu.VMEM((per,), jnp.int32),
            pltpu.VMEM((CH,), jnp.float32),
            pltpu.VMEM((zrows,), jnp.float32),
            pltpu.VMEM_SHARED((n_acc,), jnp.float32),
        ] + [pltpu.SemaphoreType.DMA] * NB,
    )
    def deg_kernel(edge_hbm, out_hbm, didx, ones_v, zbuf, acc, *ssem):
        c = lax.axis_index("c")
        s = lax.axis_index("s")
        wid = c * NS + s

        def _fill(i, _):
            zbuf[pl.ds(i * L, L)] = jnp.zeros((L,), jnp.float32)
            ones_v[pl.ds(lax.rem(i, CH // L) * L, L)] = (
                jnp.ones((L,), jnp.float32))
            return 0

        lax.fori_loop(0, zrows // L, _fill, 0)
        pltpu.sync_copy(zbuf, acc.at[pl.ds(s * zrows, zrows)])
        pltpu.sync_copy(edge_hbm.at[1, pl.ds(wid * per, per)], didx)
        plsc.subcore_barrier()

        def _start(i, b):
            pltpu.async_copy(ones_v, acc.at[didx.at[pl.ds(i * CH, CH)]],
                             ssem[b], add=True)

        def _wait(b):
            pltpu.make_async_copy(ones_v, acc.at[didx.at[pl.ds(0, CH)]],
                                  ssem[b]).wait()

        def _group(g, _):
            for b in range(NB):
                @pl.when(g > 0)
                def _():
                    _wait(b)
                _start(g * NB + b, b)
            return 0

        lax.fori_loop(0, G, _group, 0)
        for i in range(G * NB, cm):
            if i >= NB:
                _wait(i % NB)
            _start(i, i % NB)
        for j in range(max(0, cm - NB), cm):
            _wait(j % NB)
        plsc.subcore_barrier()
        pltpu.sync_copy(acc.at[pl.ds(s * zrows, zrows)],
                        out_hbm.at[c, pl.ds(s * zrows, zrows)])

    return deg_kernel


def _agg_kernel(n_acc, cm, d):
    zrows = n_acc // NS
    per = cm * CH
    G, rem = cm // NB, cm % NB
    assert rem >= GA and cm >= NB

    @functools.partial(
        pl.kernel,
        out_type=jax.ShapeDtypeStruct((NC, n_acc, d), jnp.float32),
        mesh=_sc_mesh(),
        compiler_params=_SC_PARAMS,
        scratch_types=[
            pltpu.VMEM((per,), jnp.int32),
            pltpu.VMEM((per,), jnp.int32),
        ] + [pltpu.VMEM((CH, d), jnp.float32)] * NB + [
            pltpu.VMEM((zrows, d), jnp.float32),
            pltpu.VMEM_SHARED((n_acc, d), jnp.float32),
        ] + [pltpu.SemaphoreType.DMA] * (2 * NB),
    )
    def agg_kernel(edge_hbm, table_hbm, out_hbm, sidx, didx, *rest):
        rows = rest[:NB]
        zbuf = rest[NB]
        acc = rest[NB + 1]
        gsem = rest[NB + 2:2 * NB + 2]
        ssem = rest[2 * NB + 2:]
        c = lax.axis_index("c")
        s = lax.axis_index("s")
        wid = c * NS + s

        def _fill(i, _):
            zbuf[i, :] = jnp.zeros((L,), jnp.float32)
            return 0

        lax.fori_loop(0, zrows, _fill, 0)
        pltpu.sync_copy(zbuf, acc.at[pl.ds(s * zrows, zrows)])
        pltpu.sync_copy(edge_hbm.at[0, pl.ds(wid * per, per)], sidx)
        pltpu.sync_copy(edge_hbm.at[1, pl.ds(wid * per, per)], didx)
        plsc.subcore_barrier()

        def _gather(i, b):
            pltpu.async_copy(table_hbm.at[sidx.at[pl.ds(i * CH, CH)]],
                             rows[b], gsem[b])

        def _gwait(b):
            pltpu.make_async_copy(table_hbm.at[sidx.at[pl.ds(0, CH)]],
                                  rows[b], gsem[b]).wait()

        def _scat(i, b):
            pltpu.async_copy(rows[b], acc.at[didx.at[pl.ds(i * CH, CH)]],
                             ssem[b], add=True)

        def _swait(b):
            pltpu.make_async_copy(rows[b], acc.at[didx.at[pl.ds(0, CH)]],
                                  ssem[b]).wait()

        for j in range(GA):
            _gather(j, j)

        def _group(g, _):
            for b in range(NB):
                i = g * NB + b
                tb = (b + GA) % NB
                if b + GA >= NB:
                    _swait(tb)
                    _gather(i + GA, tb)
                else:
                    @pl.when(g > 0)
                    def _():
                        _swait(tb)
                    _gather(i + GA, tb)
                _gwait(b)
                _scat(i, b)
            return 0

        lax.fori_loop(0, G, _group, 0)
        for i in range(G * NB, cm):
            b = i % NB
            tb = (b + GA) % NB
            if i + GA < cm:
                _swait(tb)
                _gather(i + GA, tb)
            _gwait(b)
            _scat(i, b)
        for j in range(max(0, cm - NB), cm):
            _swait(j % NB)
        plsc.subcore_barrier()
        pltpu.sync_copy(acc.at[pl.ds(s * zrows, zrows)],
                        out_hbm.at[c, pl.ds(s * zrows, zrows)])

    return agg_kernel


def _tc_mm(x_flat, w1_bd):
    rows, k = x_flat.shape
    d_out = w1_bd.shape[1]

    def body(x_ref, w_ref, h_ref):
        h_ref[...] = jnp.dot(x_ref[...], w_ref[...],
                             preferred_element_type=jnp.float32,
                       precision=lax.Precision.HIGHEST)

    return pl.pallas_call(
        body,
        out_shape=jax.ShapeDtypeStruct((rows, d_out), jnp.float32),
    )(x_flat, w1_bd)


def _tc_dinv(deg_view):
    rows = deg_view.shape[1]

    def body(deg_ref, dinv_ref):
        dinv_ref[...] = lax.rsqrt(1.0 + deg_ref[0] + deg_ref[1])

    return pl.pallas_call(
        body,
        out_shape=jax.ShapeDtypeStruct((rows, 128), jnp.float32),
    )(deg_view)


def _tc_scale(h1_ff, dinv_ff):
    shape = h1_ff.shape

    def body(h_ref, d_ref, hpre_ref, self1_ref):
        hp = h_ref[...] * d_ref[...]
        hpre_ref[...] = hp
        self1_ref[...] = hp * d_ref[...]

    return pl.pallas_call(
        body,
        out_shape=(jax.ShapeDtypeStruct(shape, jnp.float32),
                   jax.ShapeDtypeStruct(shape, jnp.float32)),
    )(h1_ff, dinv_ff)


def _tc_mid(agg1v, dinv_ff, self1_ff, b1t):
    shape = self1_ff.shape

    def body(agg_ref, d_ref, self1_ref, b1_ref, h2pre_ref, self2_ref):
        d = d_ref[...]
        h = d * (agg_ref[0] + agg_ref[1]) + self1_ref[...] + b1_ref[...]
        h = jnp.maximum(h, 0.0)
        h2p = h * d
        h2pre_ref[...] = h2p
        self2_ref[...] = h2p * d

    return pl.pallas_call(
        body,
        out_shape=(jax.ShapeDtypeStruct(shape, jnp.float32),
                   jax.ShapeDtypeStruct(shape, jnp.float32)),
    )(agg1v, dinv_ff, self1_ff, b1t)


def _tc_fin(agg2v, dinv_ff, self2_ff, w2_bd, b2t, blk, n_out, d_fin):
    rows = self2_ff.shape[0]

    def body(agg_ref, d_ref, self2_ref, w_ref, b_ref, out_ref):
        aggf = d_ref[...] * (agg_ref[0] + agg_ref[1]) + self2_ref[...]
        o = jnp.dot(aggf, w_ref[...], preferred_element_type=jnp.float32,
                       precision=lax.Precision.HIGHEST)
        o = o + b_ref[...]
        m = jnp.max(o, axis=1, keepdims=True)
        e = jnp.exp(o - m)
        s8 = jnp.kron(jnp.eye(blk, dtype=jnp.float32),
                      jnp.ones((blk, 1), jnp.float32))
        b8 = jnp.kron(jnp.eye(blk, dtype=jnp.float32),
                      jnp.ones((1, blk), jnp.float32))
        s = jnp.dot(e, s8, preferred_element_type=jnp.float32,
                       precision=lax.Precision.HIGHEST)
        ls = jnp.log(s)
        out_ff = (o - m) - jnp.dot(ls, b8,
                                   preferred_element_type=jnp.float32,
                                   precision=lax.Precision.HIGHEST)
        out_ref[...] = out_ff

    return pl.pallas_call(
        body,
        out_shape=jax.ShapeDtypeStruct((rows, blk * blk), jnp.float32),
    )(agg2v, dinv_ff, self2_ff, w2_bd, b2t)


def kernel(x, edge_index, W1, b1, W2, b2):
    n, d_in = x.shape
    e = edge_index.shape[1]
    d_hid = W1.shape[1]
    d_out = W2.shape[1]
    blk = 128 // d_hid

    n_acc = ((n + 1 + NS * L - 1) // (NS * L)) * (NS * L)
    frows = n_acc // blk
    e_pad = ((e + NW * CH - 1) // (NW * CH)) * (NW * CH)
    cm = e_pad // (NW * CH)

    edges = edge_index
    if e_pad != e:
        pad = jnp.concatenate(
            [jnp.zeros((1, e_pad - e), jnp.int32),
             jnp.full((1, e_pad - e), n, jnp.int32)])
        edges = jnp.concatenate([edge_index, pad], axis=1)

    x_pad = jnp.concatenate(
        [x, jnp.zeros((n_acc - n, d_in), jnp.float32)])
    x_flat = x_pad.reshape(frows, blk * d_in)
    eye = jnp.eye(blk, dtype=jnp.float32)
    w1_bd = jnp.kron(eye, W1)
    w2_pad = jnp.concatenate(
        [W2, jnp.full((d_hid, blk - d_out), 0.0)], axis=1)
    w2_bd = jnp.kron(eye, w2_pad)
    b1t = jnp.tile(b1, blk).reshape(1, 128)
    b2t = jnp.tile(jnp.concatenate(
        [b2, jnp.full((blk - d_out,), -1e9, jnp.float32)]), blk).reshape(1, -1)

    h1_ff = _tc_mm(x_flat, w1_bd)
    deg_part = _deg_kernel(n_acc, cm)(edges)
    dinv_n = _tc_dinv(deg_part.reshape(2, n_acc // 128, 128))
    dinv_ff = jnp.repeat(dinv_n.reshape(n_acc), d_hid).reshape(frows, 128)

    hpre_ff, self1_ff = _tc_scale(h1_ff, dinv_ff)
    agg1 = _agg_kernel(n_acc, cm, d_hid)(edges, hpre_ff.reshape(n_acc, d_hid))
    h2pre_ff, self2_ff = _tc_mid(agg1.reshape(2, frows, 128), dinv_ff,
                                 self1_ff, b1t)
    agg2 = _agg_kernel(n_acc, cm, d_hid)(edges, h2pre_ff.reshape(n_acc, d_hid))
    out_ff = _tc_fin(agg2.reshape(2, frows, 128), dinv_ff, self2_ff,
                     w2_bd, b2t, blk, n, d_out)
    return out_ff.reshape(n_acc, blk)[:n, :d_out]
